# baseline (speedup 1.0000x reference)
import functools

import jax
import jax.numpy as jnp
import numpy as np
from jax import lax
from jax.experimental import pallas as pl
from jax.experimental.pallas import tpu as pltpu
from jax.experimental.pallas import tpu_sc as plsc

N = 10000
E = 160000
IN_DIM = 256
OUT_DIM = 256
K = 3
HALF = 128
NCH = 1280
NCHP = 1288
EPAD = NCHP * 128
CPT = NCH // 16
NPAD = 10240
RPT = NPAD // 16
FK = K * HALF


def _build_perm() -> np.ndarray:
    perm = np.empty(K * OUT_DIM, np.int32)
    for hf in range(2):
        for k in range(K):
            for f in range(HALF):
                perm[(hf * K + k) * HALF + f] = k * OUT_DIM + hf * HALF + f
    return perm


_PERM = _build_perm()


def _proj_gauss_body(h_ref, w_ref, pt_ref, mu_ref, is_ref, hp_ref, gs_ref):
    hp_ref[...] = jnp.dot(h_ref[...], w_ref[...],
                          preferred_element_type=jnp.float32)
    p0 = pt_ref[0:1, :]
    p1 = pt_ref[1:2, :]
    for k in range(K):
        d0 = (p0 - mu_ref[k, 0]) * is_ref[k, 0]
        d1 = (p1 - mu_ref[k, 1]) * is_ref[k, 1]
        gs_ref[k:k + 1, :] = jnp.exp(-0.5 * (d0 * d0 + d1 * d1))


def _proj_gauss(h, W, pseudoT, mu, inv_sigma):
    grid = 10
    nb = N // grid
    eb = E // grid
    return pl.pallas_call(
        _proj_gauss_body,
        grid=(grid,),
        in_specs=[
            pl.BlockSpec((nb, IN_DIM), lambda i: (i, 0)),
            pl.BlockSpec((IN_DIM, K * OUT_DIM), lambda i: (0, 0)),
            pl.BlockSpec((2, eb), lambda i: (0, i)),
            pl.BlockSpec(memory_space=pltpu.SMEM),
            pl.BlockSpec(memory_space=pltpu.SMEM),
        ],
        out_specs=[
            pl.BlockSpec((nb, K * OUT_DIM), lambda i: (i, 0)),
            pl.BlockSpec((K, eb), lambda i: (0, i)),
        ],
        out_shape=[
            jax.ShapeDtypeStruct((N, K * OUT_DIM), jnp.float32),
            jax.ShapeDtypeStruct((K, E), jnp.float32),
        ],
    )(h, W, pseudoT, mu, inv_sigma)


_SC_MESH = plsc.VectorSubcoreMesh(core_axis_name="c", subcore_axis_name="s")


@functools.partial(
    pl.kernel,
    mesh=_SC_MESH,
    out_type=pltpu.HBM((2, NPAD, HALF), jnp.float32),
    scratch_types=[
        pltpu.VMEM((128,), jnp.int32),
        pltpu.VMEM((128,), jnp.int32),
        pltpu.VMEM((4, 32), jnp.int32),
        pltpu.VMEM((4, 32), jnp.int32),
        pltpu.VMEM((K, 128), jnp.float32),
        pltpu.VMEM((K, 128), jnp.float32),
        pltpu.VMEM((32,), jnp.int32),
        pltpu.VMEM((32,), jnp.int32),
        pltpu.VMEM((64, FK), jnp.float32),
        pltpu.VMEM((64, HALF), jnp.float32),
        pltpu.VMEM_SHARED((NPAD, HALF), jnp.float32),
        pltpu.SemaphoreType.DMA,
        pltpu.SemaphoreType.DMA,
        pltpu.SemaphoreType.DMA,
        pltpu.SemaphoreType.DMA,
        pltpu.SemaphoreType.DMA,
    ],
)
def _sc_agg(hp_ref, g3_ref, src_ref, dst_ref, out_ref,
            srcA, srcB, dstA, dstB, gchA, gchB, idxE, idxO, bb, msg,
            acc, smA, smB, gsE, gsO, ssem):
    cid = lax.axis_index("c")
    sid = lax.axis_index("s")

    def _zrow(i, carry):
        for f in range(HALF // 16):
            msg[i, pl.ds(f * 16, 16)] = jnp.zeros((16,), jnp.float32)
        return carry
    lax.fori_loop(0, 64, _zrow, 0)

    rbase = sid * RPT
    for j in range(RPT // 64):
        pltpu.sync_copy(msg, acc.at[pl.ds(rbase + j * 64, 64)])
    plsc.subcore_barrier()

    clo = sid * CPT
    idxs = (idxE, idxO)
    gsems = (gsE, gsO)

    def _fire_smalls(row, srcv, dstv, gch, sm):
        pltpu.async_copy(src_ref.at[row], srcv, sm)
        pltpu.async_copy(dst_ref.at[row], dstv, sm)
        pltpu.async_copy(g3_ref.at[row], gch, sm)

    def _wait_smalls(row, srcv, dstv, gch, sm):
        pltpu.make_async_copy(src_ref.at[row], srcv, sm).wait()
        pltpu.make_async_copy(dst_ref.at[row], dstv, sm).wait()
        pltpu.make_async_copy(g3_ref.at[row], gch, sm).wait()

    def _mkidx(q, srcv):
        p = q % 2
        for f in range(2):
            s = srcv[pl.ds(q * 32 + f * 16, 16)]
            idxs[p][pl.ds(f * 16, 16)] = s * 2 + cid

    def _fire_gather(q):
        p = q % 2
        return pltpu.async_copy(hp_ref.at[idxs[p]],
                                bb.at[pl.ds(p * 32, 32)], gsems[p])

    def _row(srcv, dstv, gch):
        _mkidx(0, srcv)
        descs = {0: _fire_gather(0)}
        scat = {}
        for q in range(4):
            p = q % 2
            if q < 3:
                _mkidx(q + 1, srcv)
                descs[q + 1] = _fire_gather(q + 1)
            descs[q].wait()
            if q >= 2:
                scat[q - 2].wait()

            def _go(gi, c2):
                gb = q * 32 + gi * 16
                g0v = gch[0, pl.ds(gb, 16)]
                g1v = gch[1, pl.ds(gb, 16)]
                g2v = gch[2, pl.ds(gb, 16)]

                @plsc.parallel_loop(0, 16, unroll=8)
                def _ed(t):
                    e = p * 32 + gi * 16 + t
                    sel = jnp.full((16,), t, jnp.int32)
                    ge0 = g0v.at[sel].get(mode="promise_in_bounds")
                    ge1 = g1v.at[sel].get(mode="promise_in_bounds")
                    ge2 = g2v.at[sel].get(mode="promise_in_bounds")
                    for f in range(HALF // 16):
                        sl = pl.ds(f * 16, 16)
                        msg[e, sl] = (
                            ge0 * bb[e, pl.ds(f * 16, 16)]
                            + ge1 * bb[e, pl.ds(HALF + f * 16, 16)]
                            + ge2 * bb[e, pl.ds(2 * HALF + f * 16, 16)])
                return c2
            lax.fori_loop(0, 2, _go, 0)

            scat[q] = pltpu.async_copy(msg.at[pl.ds(p * 32, 32)],
                                       acc.at[dstv.at[q]], ssem, add=True)
        scat[2].wait()
        scat[3].wait()

    _fire_smalls(clo, srcA, dstA, gchA, smA)
    _fire_smalls(clo + 1, srcB, dstB, gchB, smB)

    def _body(i, carry):
        rowA = clo + 2 * i
        _wait_smalls(rowA, srcA, dstA, gchA, smA)
        _row(srcA, dstA, gchA)
        _fire_smalls(rowA + 2, srcA, dstA, gchA, smA)
        rowB = rowA + 1
        _wait_smalls(rowB, srcB, dstB, gchB, smB)
        _row(srcB, dstB, gchB)
        _fire_smalls(rowB + 2, srcB, dstB, gchB, smB)
        return carry
    lax.fori_loop(0, CPT // 2, _body, 0)

    _wait_smalls(clo + CPT, srcA, dstA, gchA, smA)
    _wait_smalls(clo + CPT + 1, srcB, dstB, gchB, smB)

    plsc.subcore_barrier()
    pltpu.sync_copy(acc.at[pl.ds(rbase, RPT)],
                    out_ref.at[cid, pl.ds(rbase, RPT)])


def _bn_body(agg_ref, h_ref, sn_ref, gam_ref, bet_ref, bias_ref, out_ref):
    x = agg_ref[0] * sn_ref[...]
    mean = jnp.mean(x, axis=0, keepdims=True)
    xc = x - mean
    var = jnp.mean(xc * xc, axis=0, keepdims=True)
    inv = lax.rsqrt(var + 1e-5)
    out_ref[...] = (xc * inv * gam_ref[0] + bet_ref[0]
                    + bias_ref[0] + h_ref[...])


def _bn_res(agg2, h, snorm_n, gamma2, beta2, bias2):
    return pl.pallas_call(
        _bn_body,
        grid=(2,),
        in_specs=[
            pl.BlockSpec((1, N, HALF), lambda c: (c, 0, 0)),
            pl.BlockSpec((N, HALF), lambda c: (0, c)),
            pl.BlockSpec((N, 1), lambda c: (0, 0)),
            pl.BlockSpec((1, 1, HALF), lambda c: (c, 0, 0)),
            pl.BlockSpec((1, 1, HALF), lambda c: (c, 0, 0)),
            pl.BlockSpec((1, 1, HALF), lambda c: (c, 0, 0)),
        ],
        out_specs=pl.BlockSpec((N, HALF), lambda c: (0, c)),
        out_shape=jax.ShapeDtypeStruct((N, OUT_DIM), jnp.float32),
    )(agg2, h, snorm_n, gamma2, beta2, bias2)


def kernel(g, h, pseudo, snorm_n, W, mu, inv_sigma, gamma, beta, bias):
    src = g[0]
    dst = g[1]
    hp, gaussT = _proj_gauss(h, jnp.take(W, _PERM, axis=1), pseudo.T,
                             mu, inv_sigma)
    pad = EPAD - E
    srcp = jnp.pad(src, (0, pad)).reshape(NCHP, 128)
    dstp = jnp.pad(dst, (0, pad)).reshape(NCHP, 4, 32)
    g3 = jnp.pad(gaussT, ((0, 0), (0, pad))).reshape(K, NCHP, 128)
    g3 = g3.transpose(1, 0, 2)
    agg2 = _sc_agg(hp.reshape(N * 2, FK), g3, srcp, dstp)
    return _bn_res(agg2, h, snorm_n, gamma.reshape(2, 1, HALF),
                   beta.reshape(2, 1, HALF), bias.reshape(2, 1, HALF))


# speedup vs baseline: 1.0162x; 1.0162x over previous
import functools

import jax
import jax.numpy as jnp
import numpy as np
from jax import lax
from jax.experimental import pallas as pl
from jax.experimental.pallas import tpu as pltpu
from jax.experimental.pallas import tpu_sc as plsc

N = 10000
E = 160000
IN_DIM = 256
OUT_DIM = 256
K = 3
HALF = 128
NCH = 1280
NCHP = 1288
EPAD = NCHP * 128
CPT = NCH // 16
NPAD = 10240
RPT = NPAD // 16
FK = K * HALF


def _build_perm() -> np.ndarray:
    perm = np.empty(K * OUT_DIM, np.int32)
    for hf in range(2):
        for k in range(K):
            for f in range(HALF):
                perm[(hf * K + k) * HALF + f] = k * OUT_DIM + hf * HALF + f
    return perm


_PERM = _build_perm()


def _proj_gauss_body(h_ref, w_ref, pt_ref, mu_ref, is_ref, hp_ref, gs_ref):
    hp_ref[...] = jnp.dot(h_ref[...].astype(jnp.bfloat16),
                          w_ref[...].astype(jnp.bfloat16),
                          preferred_element_type=jnp.float32)
    p0 = pt_ref[0:1, :]
    p1 = pt_ref[1:2, :]
    for k in range(K):
        d0 = (p0 - mu_ref[k, 0]) * is_ref[k, 0]
        d1 = (p1 - mu_ref[k, 1]) * is_ref[k, 1]
        gs_ref[k:k + 1, :] = jnp.exp(-0.5 * (d0 * d0 + d1 * d1))


def _proj_gauss(h, W, pseudoT, mu, inv_sigma):
    grid = 10
    nb = N // grid
    eb = E // grid
    return pl.pallas_call(
        _proj_gauss_body,
        grid=(grid,),
        in_specs=[
            pl.BlockSpec((nb, IN_DIM), lambda i: (i, 0)),
            pl.BlockSpec((IN_DIM, K * OUT_DIM), lambda i: (0, 0)),
            pl.BlockSpec((2, eb), lambda i: (0, i)),
            pl.BlockSpec(memory_space=pltpu.SMEM),
            pl.BlockSpec(memory_space=pltpu.SMEM),
        ],
        out_specs=[
            pl.BlockSpec((nb, K * OUT_DIM), lambda i: (i, 0)),
            pl.BlockSpec((K, eb), lambda i: (0, i)),
        ],
        out_shape=[
            jax.ShapeDtypeStruct((N, K * OUT_DIM), jnp.float32),
            jax.ShapeDtypeStruct((K, E), jnp.float32),
        ],
    )(h, W, pseudoT, mu, inv_sigma)


_SC_MESH = plsc.VectorSubcoreMesh(core_axis_name="c", subcore_axis_name="s")


@functools.partial(
    pl.kernel,
    mesh=_SC_MESH,
    out_type=pltpu.HBM((2, NPAD, HALF), jnp.float32),
    scratch_types=[
        pltpu.VMEM((128,), jnp.int32),
        pltpu.VMEM((128,), jnp.int32),
        pltpu.VMEM((2, 64), jnp.int32),
        pltpu.VMEM((2, 64), jnp.int32),
        pltpu.VMEM((K, 128), jnp.float32),
        pltpu.VMEM((K, 128), jnp.float32),
        pltpu.VMEM((32,), jnp.int32),
        pltpu.VMEM((32,), jnp.int32),
        pltpu.VMEM((64, FK), jnp.float32),
        pltpu.VMEM((128, HALF), jnp.float32),
        pltpu.VMEM_SHARED((NPAD, HALF), jnp.float32),
        pltpu.SemaphoreType.DMA,
        pltpu.SemaphoreType.DMA,
        pltpu.SemaphoreType.DMA,
        pltpu.SemaphoreType.DMA,
        pltpu.SemaphoreType.DMA,
    ],
)
def _sc_agg(hp_ref, g3_ref, src_ref, dst_ref, out_ref,
            srcA, srcB, dstA, dstB, gchA, gchB, idxE, idxO, bb, msg,
            acc, smA, smB, gsE, gsO, ssem):
    cid = lax.axis_index("c")
    sid = lax.axis_index("s")

    def _zrow(i, carry):
        for f in range(HALF // 16):
            msg[i, pl.ds(f * 16, 16)] = jnp.zeros((16,), jnp.float32)
        return carry
    lax.fori_loop(0, 128, _zrow, 0)

    rbase = sid * RPT
    for j in range(RPT // 128):
        pltpu.sync_copy(msg, acc.at[pl.ds(rbase + j * 128, 128)])
    plsc.subcore_barrier()

    clo = sid * CPT
    idxs = (idxE, idxO)
    gsems = (gsE, gsO)

    def _fire_smalls(row, srcv, dstv, gch, sm):
        pltpu.async_copy(src_ref.at[row], srcv, sm)
        pltpu.async_copy(dst_ref.at[row], dstv, sm)
        pltpu.async_copy(g3_ref.at[row], gch, sm)

    def _wait_smalls(row, srcv, dstv, gch, sm):
        pltpu.make_async_copy(src_ref.at[row], srcv, sm).wait()
        pltpu.make_async_copy(dst_ref.at[row], dstv, sm).wait()
        pltpu.make_async_copy(g3_ref.at[row], gch, sm).wait()

    def _mkidx(q, srcv):
        p = q % 2
        for f in range(2):
            s = srcv[pl.ds(q * 32 + f * 16, 16)]
            idxs[p][pl.ds(f * 16, 16)] = s * 2 + cid

    def _fire_gather(q):
        p = q % 2
        return pltpu.async_copy(hp_ref.at[idxs[p]],
                                bb.at[pl.ds(p * 32, 32)], gsems[p])

    def _drain_scat(pred):
        @pl.when(pred)
        def _():
            pltpu.make_async_copy(msg.at[pl.ds(0, 64)],
                                  acc.at[pl.ds(rbase, 64)], ssem).wait()

    def _row(srcv, dstv, gch, can_drain):
        _mkidx(0, srcv)
        descs = {0: _fire_gather(0)}
        for q in range(4):
            p = q % 2
            if q < 3:
                _mkidx(q + 1, srcv)
                descs[q + 1] = _fire_gather(q + 1)
            descs[q].wait()
            if q % 2 == 0:
                _drain_scat(can_drain)

            def _go(gi, c2):
                gb = q * 32 + gi * 16
                g0v = gch[0, pl.ds(gb, 16)]
                g1v = gch[1, pl.ds(gb, 16)]
                g2v = gch[2, pl.ds(gb, 16)]

                @plsc.parallel_loop(0, 16, unroll=8)
                def _ed(t):
                    e = q * 32 + gi * 16 + t
                    sel = jnp.full((16,), t, jnp.int32)
                    ge0 = g0v.at[sel].get(mode="promise_in_bounds")
                    ge1 = g1v.at[sel].get(mode="promise_in_bounds")
                    ge2 = g2v.at[sel].get(mode="promise_in_bounds")
                    for f in range(HALF // 16):
                        sl = pl.ds(f * 16, 16)
                        msg[e, sl] = (
                            ge0 * bb[p * 32 + gi * 16 + t, pl.ds(f * 16, 16)]
                            + ge1 * bb[p * 32 + gi * 16 + t,
                                       pl.ds(HALF + f * 16, 16)]
                            + ge2 * bb[p * 32 + gi * 16 + t,
                                       pl.ds(2 * HALF + f * 16, 16)])
                return c2
            lax.fori_loop(0, 2, _go, 0)

            if q % 2 == 1:
                hf = q // 2
                pltpu.async_copy(msg.at[pl.ds(hf * 64, 64)],
                                 acc.at[dstv.at[hf]], ssem, add=True)

    _fire_smalls(clo, srcA, dstA, gchA, smA)
    _fire_smalls(clo + 1, srcB, dstB, gchB, smB)

    def _body(i, carry):
        rowA = clo + 2 * i
        _wait_smalls(rowA, srcA, dstA, gchA, smA)
        _row(srcA, dstA, gchA, i > 0)
        _fire_smalls(rowA + 2, srcA, dstA, gchA, smA)
        rowB = rowA + 1
        _wait_smalls(rowB, srcB, dstB, gchB, smB)
        _row(srcB, dstB, gchB, i >= 0)
        _fire_smalls(rowB + 2, srcB, dstB, gchB, smB)
        return carry
    lax.fori_loop(0, CPT // 2, _body, 0)

    _wait_smalls(clo + CPT, srcA, dstA, gchA, smA)
    _wait_smalls(clo + CPT + 1, srcB, dstB, gchB, smB)
    _drain_scat(sid >= 0)
    _drain_scat(sid >= 0)

    plsc.subcore_barrier()
    pltpu.sync_copy(acc.at[pl.ds(rbase, RPT)],
                    out_ref.at[cid, pl.ds(rbase, RPT)])


def _bn_body(agg_ref, h_ref, sn_ref, gam_ref, bet_ref, bias_ref, out_ref):
    x = agg_ref[0] * sn_ref[...]
    mean = jnp.mean(x, axis=0, keepdims=True)
    xc = x - mean
    var = jnp.mean(xc * xc, axis=0, keepdims=True)
    inv = lax.rsqrt(var + 1e-5)
    out_ref[...] = (xc * inv * gam_ref[0] + bet_ref[0]
                    + bias_ref[0] + h_ref[...])


def _bn_res(agg2, h, snorm_n, gamma2, beta2, bias2):
    return pl.pallas_call(
        _bn_body,
        grid=(2,),
        in_specs=[
            pl.BlockSpec((1, N, HALF), lambda c: (c, 0, 0)),
            pl.BlockSpec((N, HALF), lambda c: (0, c)),
            pl.BlockSpec((N, 1), lambda c: (0, 0)),
            pl.BlockSpec((1, 1, HALF), lambda c: (c, 0, 0)),
            pl.BlockSpec((1, 1, HALF), lambda c: (c, 0, 0)),
            pl.BlockSpec((1, 1, HALF), lambda c: (c, 0, 0)),
        ],
        out_specs=pl.BlockSpec((N, HALF), lambda c: (0, c)),
        out_shape=jax.ShapeDtypeStruct((N, OUT_DIM), jnp.float32),
    )(agg2, h, snorm_n, gamma2, beta2, bias2)


def kernel(g, h, pseudo, snorm_n, W, mu, inv_sigma, gamma, beta, bias):
    src = g[0]
    dst = g[1]
    hp, gaussT = _proj_gauss(h, jnp.take(W, _PERM, axis=1), pseudo.T,
                             mu, inv_sigma)
    pad = EPAD - E
    srcp = jnp.pad(src, (0, pad)).reshape(NCHP, 128)
    dstp = jnp.pad(dst, (0, pad)).reshape(NCHP, 2, 64)
    g3 = jnp.pad(gaussT, ((0, 0), (0, pad))).reshape(K, NCHP, 128)
    g3 = g3.transpose(1, 0, 2)
    agg2 = _sc_agg(hp.reshape(N * 2, FK), g3, srcp, dstp)
    return _bn_res(agg2, h, snorm_n, gamma.reshape(2, 1, HALF),
                   beta.reshape(2, 1, HALF), bias.reshape(2, 1, HALF))
